# Initial kernel scaffold; baseline (speedup 1.0000x reference)
import jax
import jax.numpy as jnp
from jax.experimental import pallas as pl


def kernel(targets, contexts, negatives, target_emb, context_emb):
    raise NotImplementedError("write your pallas kernel here")


# speedup vs baseline: 3.6268x; 3.6268x over previous
import functools

import jax
import jax.numpy as jnp
import numpy as np
from jax import lax
from jax.experimental import pallas as pl
from jax.experimental.pallas import tpu as pltpu
from jax.experimental.pallas import tpu_sc as plsc


def _sc_scores(tgt, ctx, neg_flat, target_emb, context_emb, B, K, D):
    info = plsc.get_sparse_core_info()
    NC, NS, L = info.num_cores, info.num_subcores, info.num_lanes
    NW = NC * NS
    bpw = B // NW
    C = 64
    nch = bpw // C
    G = C // L
    NG = 128
    ngath = (C * K) // NG

    mesh = plsc.VectorSubcoreMesh(core_axis_name="c", subcore_axis_name="s")

    @functools.partial(
        pl.kernel,
        mesh=mesh,
        out_type=[
            jax.ShapeDtypeStruct((NW, bpw), jnp.float32),
            jax.ShapeDtypeStruct((NW, K, bpw), jnp.float32),
        ],
        scratch_types=[
            pltpu.VMEM((C,), jnp.int32),
            pltpu.VMEM((C,), jnp.int32),
            pltpu.VMEM((C * K,), jnp.int32),
            pltpu.VMEM((C, D), jnp.float32),
            pltpu.VMEM((C, D), jnp.float32),
            pltpu.VMEM((C * K, D), jnp.float32),
            pltpu.VMEM((bpw,), jnp.float32),
            pltpu.VMEM((K, bpw), jnp.float32),
            pltpu.SemaphoreType.DMA,
        ],
        compiler_params=pltpu.CompilerParams(
            needs_layout_passes=False, use_tc_tiling_on_sc=False),
    )
    def sc_scores(tgt_h, ctx_h, negf_h, temb_h, cemb_h, pos_h, neg_h,
                  idx_t, idx_c, idx_n, vt, vc, vn, pos_buf, neg_buf, sem):
        wid = lax.axis_index("s") * NC + lax.axis_index("c")
        base = wid * bpw
        lane = lax.iota(jnp.int32, L)
        zero = jnp.zeros((L,), jnp.float32)

        def chunk_body(ci, carry):
            off = base + ci * C
            pltpu.sync_copy(tgt_h.at[pl.ds(off, C)], idx_t)
            pltpu.sync_copy(ctx_h.at[pl.ds(off, C)], idx_c)
            pltpu.sync_copy(negf_h.at[pl.ds(off * K, C * K)], idx_n)
            cps = [
                pltpu.async_copy(temb_h.at[idx_t], vt, sem),
                pltpu.async_copy(cemb_h.at[idx_c], vc, sem),
            ]
            for j in range(ngath):
                cps.append(pltpu.async_copy(
                    cemb_h.at[idx_n.at[pl.ds(j * NG, NG)]],
                    vn.at[pl.ds(j * NG, NG)], sem))
            for cp in cps:
                cp.wait()

            for g in range(G):
                row0 = ci * C + g * L
                b_idx = lane + g * L
                n_row0 = (lane + g * L) * K
                pos_buf[pl.ds(row0, L)] = zero
                for k in range(K):
                    neg_buf[k, pl.ds(row0, L)] = zero

                def d_body(d, carry2):
                    dd = jnp.full((L,), d, dtype=jnp.int32)
                    vtd = plsc.load_gather(vt, [b_idx, dd])
                    vcd = plsc.load_gather(vc, [b_idx, dd])
                    plsc.addupdate(pos_buf.at[pl.ds(row0, L)], vtd * vcd)
                    for k in range(K):
                        vnd = plsc.load_gather(vn, [n_row0 + k, dd])
                        plsc.addupdate(neg_buf.at[k, pl.ds(row0, L)],
                                       vnd * vtd)
                    return carry2

                lax.fori_loop(0, D, d_body, 0)
            return carry

        lax.fori_loop(0, nch, chunk_body, 0)
        pltpu.sync_copy(pos_buf, pos_h.at[wid])
        pltpu.sync_copy(neg_buf, neg_h.at[wid])

    return sc_scores(tgt, ctx, neg_flat, target_emb, context_emb)


def _tc_loss(pos2, neg2, B):
    def tc_body(p_ref, n_ref, o_ref):
        p = p_ref[...]
        n = n_ref[...]
        lsp = jnp.minimum(p, 0.0) - jnp.log(1.0 + jnp.exp(-jnp.abs(p)))
        lsn = jnp.minimum(-n, 0.0) - jnp.log(1.0 + jnp.exp(-jnp.abs(n)))
        o_ref[0, 0] = -(jnp.sum(lsp) + jnp.sum(lsn)) * np.float32(1.0 / B)

    return pl.pallas_call(
        tc_body,
        out_shape=jax.ShapeDtypeStruct((1, 1), jnp.float32),
        out_specs=pl.BlockSpec(memory_space=pltpu.SMEM),
    )(pos2, neg2)


def kernel(targets, contexts, negatives, target_emb, context_emb):
    B, = targets.shape
    _, K = negatives.shape
    _, D = target_emb.shape

    tgt = targets.astype(jnp.int32)
    ctx = contexts.astype(jnp.int32)
    neg_flat = negatives.astype(jnp.int32).reshape(-1)

    pos, neg = _sc_scores(tgt, ctx, neg_flat, target_emb, context_emb, B, K, D)

    pos2 = pos.reshape(B // 128, 128)
    neg2 = neg.reshape(-1, 128)
    loss = _tc_loss(pos2, neg2, B)
    return loss[0, 0]
